# baseline (speedup 1.0000x reference)
import functools

import jax
import jax.numpy as jnp
from jax import lax
from jax.experimental import pallas as pl
from jax.experimental.pallas import tpu as pltpu
from jax.experimental.pallas import tpu_sc as plsc

_N = 10000
_D = 128
_E = 320000

_NC = 2
_NS = 16
_NW = _NC * _NS

_NPAD = 10112
_TS = _NPAD // _NS

_CHUNK = 384
_G = _CHUNK // 128
_NCHUNKS = 27
_EPW = _CHUNK * _NCHUNKS
_EPAD = _EPW * _NW
_IDXROWS_PER_W = _EPW // 128


def _mesh():
    return plsc.VectorSubcoreMesh(
        core_axis_name="c", subcore_axis_name="s",
        num_cores=_NC, num_subcores=_NS)


def _deg_body(idx_hbm, ones_hbm, zeros_hbm, out_hbm, idx_v, ones_v, hist_sh):
    c = lax.axis_index("c")
    s = lax.axis_index("s")
    wid = s * _NC + c
    pltpu.sync_copy(ones_hbm, ones_v)
    pltpu.sync_copy(zeros_hbm, hist_sh.at[pl.ds(s * _TS, _TS)])
    plsc.subcore_barrier()

    def chunk(k, carry):
        cid = wid * _NCHUNKS + k
        pltpu.sync_copy(idx_hbm.at[cid], idx_v)
        for j in range(_G):
            pltpu.sync_copy(ones_v, hist_sh.at[idx_v.at[2 * j]], add=True)
        return carry

    lax.fori_loop(0, _NCHUNKS, chunk, 0)
    plsc.subcore_barrier()
    pltpu.sync_copy(hist_sh.at[pl.ds(s * _TS, _TS)],
                    out_hbm.at[c, pl.ds(s * _TS, _TS)])


@functools.lru_cache(maxsize=None)
def _deg_call():
    return pl.kernel(
        _deg_body,
        out_type=jax.ShapeDtypeStruct((_NC, _NPAD, _D), jnp.float32),
        mesh=_mesh(),
        scratch_types=[
            pltpu.VMEM((2 * _G, 128), jnp.int32),
            pltpu.VMEM((128, _D), jnp.float32),
            pltpu.VMEM_SHARED((_NPAD, _D), jnp.float32),
        ],
    )


def _hop_body(y_hbm, idx_hbm, out_hbm, idx_v, rows_v, acc_sh, gsem, ssem):
    c = lax.axis_index("c")
    s = lax.axis_index("s")
    wid = s * _NC + c
    pltpu.sync_copy(y_hbm.at[pl.ds(s * _TS, _TS)],
                    acc_sh.at[pl.ds(s * _TS, _TS)])
    plsc.subcore_barrier()

    def chunk(k, carry):
        cid = wid * _NCHUNKS + k
        pltpu.sync_copy(idx_hbm.at[cid], idx_v)
        gh = [
            pltpu.async_copy(y_hbm.at[idx_v.at[2 * j + 1]],
                             rows_v.at[j], gsem)
            for j in range(_G)
        ]
        for h in gh:
            h.wait()
        sh = [
            pltpu.async_copy(rows_v.at[j], acc_sh.at[idx_v.at[2 * j]],
                             ssem, add=True)
            for j in range(_G)
        ]
        for h in sh:
            h.wait()
        return carry

    lax.fori_loop(0, _NCHUNKS, chunk, 0)
    plsc.subcore_barrier()
    pltpu.sync_copy(acc_sh.at[pl.ds(s * _TS, _TS)],
                    out_hbm.at[c, pl.ds(s * _TS, _TS)])


@functools.lru_cache(maxsize=None)
def _hop_call():
    return pl.kernel(
        _hop_body,
        out_type=jax.ShapeDtypeStruct((_NC, _NPAD, _D), jnp.float32),
        mesh=_mesh(),
        scratch_types=[
            pltpu.VMEM((2 * _G, 128), jnp.int32),
            pltpu.VMEM((_G, 128, _D), jnp.float32),
            pltpu.VMEM_SHARED((_NPAD, _D), jnp.float32),
            pltpu.SemaphoreType.DMA,
            pltpu.SemaphoreType.DMA,
        ],
    )


def _deg_from_partials(degp):
    deg = degp[0, :, 0:1] + degp[1, :, 0:1] + 1.0
    return deg


def _prep_tc(x_ref, degp_ref, y0_ref):
    x = x_ref[...]
    m = jnp.mean(x, axis=0, keepdims=True)
    xc = x - m
    var = jnp.sum(xc * xc, axis=0, keepdims=True) / (_N - 1)
    std = jnp.sqrt(var)
    std = jnp.where(std == 0.0, 1.0, std)
    xn = xc / std
    xn = jnp.concatenate([xn, jnp.zeros((_NPAD - _N, _D), jnp.float32)], axis=0)
    deg = _deg_from_partials(degp_ref[...])
    y0_ref[...] = xn * lax.rsqrt(deg)


def _mid_tc(p_ref, y_ref, degp_ref, o_ref):
    t = p_ref[0] + p_ref[1] - y_ref[...]
    deg = _deg_from_partials(degp_ref[...])
    o_ref[...] = t / deg


def _fin_tc(p_ref, y_ref, degp_ref, o_ref):
    t = p_ref[0] + p_ref[1] - y_ref[...]
    deg = _deg_from_partials(degp_ref[...])
    o_ref[...] = (t * lax.rsqrt(deg))[:_N]


_prep_call = pl.pallas_call(
    _prep_tc, out_shape=jax.ShapeDtypeStruct((_NPAD, _D), jnp.float32))
_mid_call = pl.pallas_call(
    _mid_tc, out_shape=jax.ShapeDtypeStruct((_NPAD, _D), jnp.float32))
_fin_call = pl.pallas_call(
    _fin_tc, out_shape=jax.ShapeDtypeStruct((_N, _D), jnp.float32))


@jax.jit
def kernel(x, edge_index):
    row = edge_index[0]
    col = edge_index[1]
    npad_e = _EPAD - _E
    row_p = jnp.concatenate(
        [row, jnp.full((npad_e,), _N, jnp.int32)]).reshape(-1, _G, 128)
    col_p = jnp.concatenate(
        [col, jnp.zeros((npad_e,), jnp.int32)]).reshape(-1, _G, 128)
    idx_p = jnp.stack([row_p, col_p], axis=2).reshape(-1, 2 * _G, 128)

    ones = jnp.ones((128, _D), jnp.float32)
    zeros = jnp.zeros((_TS, _D), jnp.float32)

    degp = _deg_call()(idx_p, ones, zeros)
    y0 = _prep_call(x, degp)
    p1 = _hop_call()(y0, idx_p)
    y1 = _mid_call(p1, y0, degp)
    p2 = _hop_call()(y1, idx_p)
    return _fin_call(p2, y1, degp)


# speedup vs baseline: 1.1447x; 1.1447x over previous
import functools

import jax
import jax.numpy as jnp
from jax import lax
from jax.experimental import pallas as pl
from jax.experimental.pallas import tpu as pltpu
from jax.experimental.pallas import tpu_sc as plsc

_N = 10000
_D = 128
_E = 320000

_NC = 2
_NS = 16
_NW = _NC * _NS

_NPAD = 10240
_NY = 10112
_TSY = _NPAD // _NS
_TSG = _NY // _NS

_CHUNK = 256
_G = _CHUNK // 128
_NCHUNKS_TOT = 1280
_EPAD = _NCHUNKS_TOT * _CHUNK
_NCH = _NCHUNKS_TOT // _NW


def _mesh():
    return plsc.VectorSubcoreMesh(
        core_axis_name="c", subcore_axis_name="s",
        num_cores=_NC, num_subcores=_NS)


def _deg_body(idx_hbm, ones_hbm, zeros_hbm, out_hbm, idx_v, ones_v, hist_sh):
    c = lax.axis_index("c")
    s = lax.axis_index("s")
    wid = s * _NC + c
    pltpu.sync_copy(ones_hbm, ones_v)
    pltpu.sync_copy(zeros_hbm, hist_sh.at[pl.ds(s * _TSY, _TSY)])
    plsc.subcore_barrier()

    def chunk(k, carry):
        cid = wid * _NCH + k
        pltpu.sync_copy(idx_hbm.at[cid], idx_v)
        for j in range(_G):
            pltpu.sync_copy(ones_v, hist_sh.at[idx_v.at[j]], add=True)
        return carry

    lax.fori_loop(0, _NCH, chunk, 0)
    plsc.subcore_barrier()
    pltpu.sync_copy(hist_sh.at[pl.ds(s * _TSY, _TSY)],
                    out_hbm.at[c, pl.ds(s * _TSY, _TSY)])


@functools.lru_cache(maxsize=None)
def _deg_call():
    return pl.kernel(
        _deg_body,
        out_type=jax.ShapeDtypeStruct((_NC, _NPAD, _D), jnp.float32),
        mesh=_mesh(),
        scratch_types=[
            pltpu.VMEM((_G, 128), jnp.int32),
            pltpu.VMEM((128, _D), jnp.float32),
            pltpu.VMEM_SHARED((_NPAD, _D), jnp.float32),
        ],
    )


def _hop_body(y_hbm, idx_hbm, out_hbm, idx_v, rows_v, acc_sh, gsem, ssem):
    c = lax.axis_index("c")
    s = lax.axis_index("s")
    wid = s * _NC + c
    pltpu.sync_copy(y_hbm.at[pl.ds(s * _TSG, _TSG)],
                    acc_sh.at[pl.ds(s * _TSG, _TSG)])
    plsc.subcore_barrier()

    def chunk(k, carry):
        cid = wid * _NCH + k
        pltpu.sync_copy(idx_hbm.at[cid], idx_v)
        gh = [
            pltpu.async_copy(y_hbm.at[idx_v.at[2 * j + 1]],
                             rows_v.at[j], gsem)
            for j in range(_G)
        ]
        for h in gh:
            h.wait()
        sh = [
            pltpu.async_copy(rows_v.at[j], acc_sh.at[idx_v.at[2 * j]],
                             ssem, add=True)
            for j in range(_G)
        ]
        for h in sh:
            h.wait()
        return carry

    lax.fori_loop(0, _NCH, chunk, 0)
    plsc.subcore_barrier()
    pltpu.sync_copy(acc_sh.at[pl.ds(s * _TSG, _TSG)],
                    out_hbm.at[c, pl.ds(s * _TSG, _TSG)])


@functools.lru_cache(maxsize=None)
def _hop_call():
    return pl.kernel(
        _hop_body,
        out_type=jax.ShapeDtypeStruct((_NC, _NY, _D), jnp.float32),
        mesh=_mesh(),
        scratch_types=[
            pltpu.VMEM((2 * _G, 128), jnp.int32),
            pltpu.VMEM((_G, 128, _D), jnp.float32),
            pltpu.VMEM_SHARED((_NY, _D), jnp.float32),
            pltpu.SemaphoreType.DMA,
            pltpu.SemaphoreType.DMA,
        ],
    )


def _deg_from_partials(degp):
    return degp[0, :_NY, 0:1] + degp[1, :_NY, 0:1] + 1.0


def _prep_tc(x_ref, degp_ref, y0_ref):
    x = x_ref[...]
    m = jnp.mean(x, axis=0, keepdims=True)
    xc = x - m
    var = jnp.sum(xc * xc, axis=0, keepdims=True) / (_N - 1)
    std = jnp.sqrt(var)
    std = jnp.where(std == 0.0, 1.0, std)
    xn = xc / std
    xn = jnp.concatenate([xn, jnp.zeros((_NY - _N, _D), jnp.float32)], axis=0)
    deg = _deg_from_partials(degp_ref[...])
    y0_ref[...] = xn * lax.rsqrt(deg)


def _mid_tc(p_ref, y_ref, degp_ref, o_ref):
    deg = _deg_from_partials(degp_ref[...])
    t = p_ref[0] + p_ref[1] - y_ref[...]
    o_ref[...] = t / deg


def _fin_tc(p_ref, y_ref, degp_ref, o_ref):
    deg = _deg_from_partials(degp_ref[...])
    t = p_ref[0] + p_ref[1] - y_ref[...]
    o_ref[...] = (t * lax.rsqrt(deg))[:_N]


_prep_call = pl.pallas_call(
    _prep_tc, out_shape=jax.ShapeDtypeStruct((_NY, _D), jnp.float32))
_mid_call = pl.pallas_call(
    _mid_tc, out_shape=jax.ShapeDtypeStruct((_NY, _D), jnp.float32))
_fin_call = pl.pallas_call(
    _fin_tc, out_shape=jax.ShapeDtypeStruct((_N, _D), jnp.float32))


@jax.jit
def kernel(x, edge_index):
    row = edge_index[0]
    col = edge_index[1]
    npad_e = _EPAD - _E
    row_p = jnp.concatenate([row, jnp.full((npad_e,), _N, jnp.int32)])
    col_p = jnp.concatenate([col, jnp.zeros((npad_e,), jnp.int32)])
    row_g = row_p.reshape(-1, _G, 128)
    col_g = col_p.reshape(-1, _G, 128)
    idx_hop = jnp.stack([row_g, col_g], axis=2).reshape(-1, 2 * _G, 128)

    ones = jnp.ones((128, _D), jnp.float32)
    zeros = jnp.zeros((_TSY, _D), jnp.float32)

    degp = _deg_call()(row_g, ones, zeros)
    y0 = _prep_call(x, degp)
    p1 = _hop_call()(y0, idx_hop)
    y1 = _mid_call(p1, y0, degp)
    p2 = _hop_call()(y1, idx_hop)
    return _fin_call(p2, y1, degp)


# speedup vs baseline: 1.7328x; 1.5137x over previous
import functools

import jax
import jax.numpy as jnp
from jax import lax
from jax.experimental import pallas as pl
from jax.experimental.pallas import tpu as pltpu
from jax.experimental.pallas import tpu_sc as plsc

_N = 10000
_D = 128
_E = 320000

_NC = 2
_NS = 16
_NW = _NC * _NS

_NPAD = 10240
_NY = 10112
_TSY = _NPAD // _NS
_TSG = _NY // _NS

_CHUNK = 256
_G = _CHUNK // 128
_NCHUNKS_TOT = 1280
_EPAD = _NCHUNKS_TOT * _CHUNK
_NCH = _NCHUNKS_TOT // _NW


def _mesh():
    return plsc.VectorSubcoreMesh(
        core_axis_name="c", subcore_axis_name="s",
        num_cores=_NC, num_subcores=_NS)


def _deg_body(idx_hbm, ones_hbm, zeros_hbm, out_hbm, idx_v, ones_v, hist_sh):
    c = lax.axis_index("c")
    s = lax.axis_index("s")
    wid = s * _NC + c
    pltpu.sync_copy(ones_hbm, ones_v)
    pltpu.sync_copy(zeros_hbm, hist_sh.at[pl.ds(s * _TSY, _TSY)])
    plsc.subcore_barrier()

    def chunk(k, carry):
        cid = wid * _NCH + k
        pltpu.sync_copy(idx_hbm.at[cid], idx_v)
        for j in range(_G):
            pltpu.sync_copy(ones_v, hist_sh.at[idx_v.at[j]], add=True)
        return carry

    lax.fori_loop(0, _NCH, chunk, 0)
    plsc.subcore_barrier()
    pltpu.sync_copy(hist_sh.at[pl.ds(s * _TSY, _TSY)],
                    out_hbm.at[c, pl.ds(s * _TSY, _TSY)])


@functools.lru_cache(maxsize=None)
def _deg_call():
    return pl.kernel(
        _deg_body,
        out_type=jax.ShapeDtypeStruct((_NC, _NPAD, _D), jnp.float32),
        mesh=_mesh(),
        scratch_types=[
            pltpu.VMEM((_G, 128), jnp.int32),
            pltpu.VMEM((128, _D), jnp.float32),
            pltpu.VMEM_SHARED((_NPAD, _D), jnp.float32),
        ],
    )


def _hop_body(y_hbm, idx_hbm, out_hbm, idx_v, rows_v, acc_sh, gsem, ssem):
    c = lax.axis_index("c")
    s = lax.axis_index("s")
    wid = s * _NC + c
    pltpu.sync_copy(y_hbm.at[pl.ds(s * _TSG, _TSG)],
                    acc_sh.at[pl.ds(s * _TSG, _TSG)])
    plsc.subcore_barrier()

    def chunk(k, carry):
        cid = wid * _NCH + k
        pltpu.sync_copy(idx_hbm.at[cid], idx_v)
        gh = [
            pltpu.async_copy(y_hbm.at[idx_v.at[2 * j + 1]],
                             rows_v.at[j], gsem)
            for j in range(_G)
        ]
        for h in gh:
            h.wait()
        sh = [
            pltpu.async_copy(rows_v.at[j], acc_sh.at[idx_v.at[2 * j]],
                             ssem, add=True)
            for j in range(_G)
        ]
        for h in sh:
            h.wait()
        return carry

    lax.fori_loop(0, _NCH, chunk, 0)
    plsc.subcore_barrier()
    pltpu.sync_copy(acc_sh.at[pl.ds(s * _TSG, _TSG)],
                    out_hbm.at[c, pl.ds(s * _TSG, _TSG)])


@functools.lru_cache(maxsize=None)
def _hop_call():
    return pl.kernel(
        _hop_body,
        out_type=jax.ShapeDtypeStruct((_NC, _NY, _D), jnp.float32),
        mesh=_mesh(),
        scratch_types=[
            pltpu.VMEM((2 * _G, 128), jnp.int32),
            pltpu.VMEM((_G, 128, _D), jnp.float32),
            pltpu.VMEM_SHARED((_NY, _D), jnp.float32),
            pltpu.SemaphoreType.DMA,
            pltpu.SemaphoreType.DMA,
        ],
    )


def _deg_from_partials(degp):
    return degp[0, :_NY, 0:1] + degp[1, :_NY, 0:1] + 1.0


def _prep_tc(x_ref, degp_ref, y0_ref):
    x = x_ref[...]
    m = jnp.mean(x, axis=0, keepdims=True)
    xc = x - m
    var = jnp.sum(xc * xc, axis=0, keepdims=True) / (_N - 1)
    std = jnp.sqrt(var)
    std = jnp.where(std == 0.0, 1.0, std)
    xn = xc / std
    xn = jnp.concatenate([xn, jnp.zeros((_NY - _N, _D), jnp.float32)], axis=0)
    deg = _deg_from_partials(degp_ref[...])
    y0_ref[...] = xn * lax.rsqrt(deg)


def _mid_tc(p_ref, y_ref, degp_ref, o_ref):
    deg = _deg_from_partials(degp_ref[...])
    t = p_ref[0] + p_ref[1] - y_ref[...]
    o_ref[...] = t / deg


def _fin_tc(p_ref, y_ref, degp_ref, o_ref):
    deg = _deg_from_partials(degp_ref[...])
    t = p_ref[0] + p_ref[1] - y_ref[...]
    o_ref[...] = (t * lax.rsqrt(deg))[:_N]


_prep_call = pl.pallas_call(
    _prep_tc, out_shape=jax.ShapeDtypeStruct((_NY, _D), jnp.float32))
_mid_call = pl.pallas_call(
    _mid_tc, out_shape=jax.ShapeDtypeStruct((_NY, _D), jnp.float32))
_fin_call = pl.pallas_call(
    _fin_tc, out_shape=jax.ShapeDtypeStruct((_N, _D), jnp.float32))


@jax.jit
def kernel(x, edge_index):
    row = edge_index[0]
    col = edge_index[1]
    npad_e = _EPAD - _E
    row_p = jnp.concatenate([row, jnp.full((npad_e,), _N, jnp.int32)])
    col_p = jnp.concatenate([col, jnp.zeros((npad_e,), jnp.int32)])
    row_p = row_p.reshape(256, 1280).T.reshape(-1)
    col_p = col_p.reshape(256, 1280).T.reshape(-1)
    row_g = row_p.reshape(-1, _G, 128)
    col_g = col_p.reshape(-1, _G, 128)
    idx_hop = jnp.stack([row_g, col_g], axis=2).reshape(-1, 2 * _G, 128)

    ones = jnp.ones((128, _D), jnp.float32)
    zeros = jnp.zeros((_TSY, _D), jnp.float32)

    degp = _deg_call()(row_g, ones, zeros)
    y0 = _prep_call(x, degp)
    p1 = _hop_call()(y0, idx_hop)
    y1 = _mid_call(p1, y0, degp)
    p2 = _hop_call()(y1, idx_hop)
    return _fin_call(p2, y1, degp)


# speedup vs baseline: 1.7353x; 1.0014x over previous
import functools

import jax
import jax.numpy as jnp
from jax import lax
from jax.experimental import pallas as pl
from jax.experimental.pallas import tpu as pltpu
from jax.experimental.pallas import tpu_sc as plsc

_N = 10000
_D = 128
_E = 320000

_NC = 2
_NS = 16
_NW = _NC * _NS

_NPAD = 10240
_NY = 10112
_TSY = _NPAD // _NS
_TSG = _NY // _NS

_CHUNK = 256
_G = _CHUNK // 128
_NCHUNKS_TOT = 1280
_EPAD = _NCHUNKS_TOT * _CHUNK
_NCH = _NCHUNKS_TOT // _NW


def _mesh():
    return plsc.VectorSubcoreMesh(
        core_axis_name="c", subcore_axis_name="s",
        num_cores=_NC, num_subcores=_NS)


def _deg_body(idx_hbm, ones_hbm, zeros_hbm, out_hbm, idx_v, ones_v, hist_sh):
    c = lax.axis_index("c")
    s = lax.axis_index("s")
    wid = s * _NC + c
    pltpu.sync_copy(ones_hbm, ones_v)
    pltpu.sync_copy(zeros_hbm, hist_sh.at[pl.ds(s * _TSY, _TSY)])
    plsc.subcore_barrier()

    def chunk(k, carry):
        cid = wid * _NCH + k
        pltpu.sync_copy(idx_hbm.at[cid], idx_v)
        for j in range(_G):
            pltpu.sync_copy(ones_v, hist_sh.at[idx_v.at[j]], add=True)
        return carry

    lax.fori_loop(0, _NCH, chunk, 0)
    plsc.subcore_barrier()
    pltpu.sync_copy(hist_sh.at[pl.ds(s * _TSY, _TSY)],
                    out_hbm.at[c, pl.ds(s * _TSY, _TSY)])


@functools.lru_cache(maxsize=None)
def _deg_call():
    return pl.kernel(
        _deg_body,
        out_type=jax.ShapeDtypeStruct((_NC, _NPAD, _D), jnp.float32),
        mesh=_mesh(),
        scratch_types=[
            pltpu.VMEM((_G, 128), jnp.int32),
            pltpu.VMEM((128, _D), jnp.float32),
            pltpu.VMEM_SHARED((_NPAD, _D), jnp.float32),
        ],
    )


def _hop_body(y_hbm, idx_hbm, out_hbm, idx_v, rows_v, acc_sh, gsem, ssem):
    c = lax.axis_index("c")
    s = lax.axis_index("s")
    wid = s * _NC + c
    pltpu.sync_copy(y_hbm.at[pl.ds(s * _TSG, _TSG)],
                    acc_sh.at[pl.ds(s * _TSG, _TSG)])
    plsc.subcore_barrier()

    def chunk(k, carry):
        cid = wid * _NCH + k
        pltpu.sync_copy(idx_hbm.at[cid], idx_v)
        gh = [
            pltpu.async_copy(y_hbm.at[idx_v.at[2 * j + 1]],
                             rows_v.at[j], gsem)
            for j in range(_G)
        ]
        sh = []
        for j in range(_G):
            gh[j].wait()
            sh.append(
                pltpu.async_copy(rows_v.at[j], acc_sh.at[idx_v.at[2 * j]],
                                 ssem, add=True))
        for h in sh:
            h.wait()
        return carry

    lax.fori_loop(0, _NCH, chunk, 0)
    plsc.subcore_barrier()
    pltpu.sync_copy(acc_sh.at[pl.ds(s * _TSG, _TSG)],
                    out_hbm.at[c, pl.ds(s * _TSG, _TSG)])


@functools.lru_cache(maxsize=None)
def _hop_call():
    return pl.kernel(
        _hop_body,
        out_type=jax.ShapeDtypeStruct((_NC, _NY, _D), jnp.float32),
        mesh=_mesh(),
        scratch_types=[
            pltpu.VMEM((2 * _G, 128), jnp.int32),
            pltpu.VMEM((_G, 128, _D), jnp.float32),
            pltpu.VMEM_SHARED((_NY, _D), jnp.float32),
            pltpu.SemaphoreType.DMA,
            pltpu.SemaphoreType.DMA,
        ],
    )


def _deg_from_partials(degp):
    return degp[0, :_NY, 0:1] + degp[1, :_NY, 0:1] + 1.0


def _prep_tc(x_ref, degp_ref, y0_ref):
    x = x_ref[...]
    m = jnp.mean(x, axis=0, keepdims=True)
    xc = x - m
    var = jnp.sum(xc * xc, axis=0, keepdims=True) / (_N - 1)
    std = jnp.sqrt(var)
    std = jnp.where(std == 0.0, 1.0, std)
    xn = xc / std
    xn = jnp.concatenate([xn, jnp.zeros((_NY - _N, _D), jnp.float32)], axis=0)
    deg = _deg_from_partials(degp_ref[...])
    y0_ref[...] = xn * lax.rsqrt(deg)


def _mid_tc(p_ref, y_ref, degp_ref, o_ref):
    deg = _deg_from_partials(degp_ref[...])
    t = p_ref[0] + p_ref[1] - y_ref[...]
    o_ref[...] = t / deg


def _fin_tc(p_ref, y_ref, degp_ref, o_ref):
    deg = _deg_from_partials(degp_ref[...])
    t = p_ref[0] + p_ref[1] - y_ref[...]
    o_ref[...] = (t * lax.rsqrt(deg))[:_N]


_prep_call = pl.pallas_call(
    _prep_tc, out_shape=jax.ShapeDtypeStruct((_NY, _D), jnp.float32))
_mid_call = pl.pallas_call(
    _mid_tc, out_shape=jax.ShapeDtypeStruct((_NY, _D), jnp.float32))
_fin_call = pl.pallas_call(
    _fin_tc, out_shape=jax.ShapeDtypeStruct((_N, _D), jnp.float32))


@jax.jit
def kernel(x, edge_index):
    row = edge_index[0]
    col = edge_index[1]
    npad_e = _EPAD - _E
    row_p = jnp.concatenate([row, jnp.full((npad_e,), _N, jnp.int32)])
    col_p = jnp.concatenate([col, jnp.zeros((npad_e,), jnp.int32)])
    row_p = row_p.reshape(256, 1280).T.reshape(-1)
    col_p = col_p.reshape(256, 1280).T.reshape(-1)
    row_g = row_p.reshape(-1, _G, 128)
    col_g = col_p.reshape(-1, _G, 128)
    idx_hop = jnp.stack([row_g, col_g], axis=2).reshape(-1, 2 * _G, 128)

    ones = jnp.ones((128, _D), jnp.float32)
    zeros = jnp.zeros((_TSY, _D), jnp.float32)

    degp = _deg_call()(row_g, ones, zeros)
    y0 = _prep_call(x, degp)
    p1 = _hop_call()(y0, idx_hop)
    y1 = _mid_call(p1, y0, degp)
    p2 = _hop_call()(y1, idx_hop)
    return _fin_call(p2, y1, degp)
